# Initial kernel scaffold; baseline (speedup 1.0000x reference)
import jax
import jax.numpy as jnp
from jax.experimental import pallas as pl


def kernel(emb, h, c, W_iou, U_iou, b_iou, W_f, U_f, b_f, g0_s, g0_b, g1_s, g1_b, g2_s, g2_b, gf_s, gf_b, gc_s, gc_b, n_type, edge_index):
    raise NotImplementedError("write your pallas kernel here")


# speedup vs baseline: 1.2862x; 1.2862x over previous
import functools

import jax
import jax.numpy as jnp
from jax import lax
from jax.experimental import pallas as pl
from jax.experimental.pallas import tpu as pltpu
from jax.experimental.pallas import tpu_sc as plsc

N = 10000
H = 128
T = 10240
ROWS = 2 * T
NSWEEP = 2
OWN = 320
ACCR = OWN + 8
DUMMY = OWN
EB = 1024
NBLK = 314
E_PAD = NBLK * EB
BATCH = 128
R = 2560
GRID = 4


def _sc_segsum(hc, src_p, dst_p, ntype):
    mesh = plsc.VectorSubcoreMesh(core_axis_name="c", subcore_axis_name="s")

    @functools.partial(
        pl.kernel,
        out_type=jax.ShapeDtypeStruct((ROWS, 2 * H), jnp.float32),
        mesh=mesh,
        compiler_params=pltpu.CompilerParams(needs_layout_passes=False),
        scratch_types=[
            pltpu.VMEM((N,), jnp.int32),
            pltpu.VMEM((EB,), jnp.int32),
            pltpu.VMEM((EB,), jnp.int32),
            pltpu.VMEM((160,), jnp.int32),
            pltpu.VMEM((160,), jnp.int32),
            pltpu.VMEM((BATCH, 2 * H), jnp.float32),
            pltpu.VMEM((ACCR, 2 * H), jnp.float32),
            pltpu.SemaphoreType.DMA,
        ],
    )
    def k(hc_hbm, src_hbm, dst_hbm, nt_hbm, out_hbm,
          ntype_v, src_blk, dst_blk, comp_row, comp_src, rows_v, acc, sem):
        c = lax.axis_index("c")
        s = lax.axis_index("s")
        wid = c * 16 + s
        pltpu.sync_copy(nt_hbm, ntype_v)
        lanes = lax.iota(jnp.int32, 16)
        zv = jnp.zeros((16,), jnp.float32)

        def accumulate(gmask_rows):
            for g in range(BATCH // 16):
                ev = g * 16 + lanes
                rv = comp_row[pl.ds(g * 16, 16)]
                if gmask_rows is not None:
                    rv = jnp.where(gmask_rows(g), rv, DUMMY + (lanes & 7))

                def kb(kk, carry):
                    for u in range(4):
                        cv = kk * 4 + u
                        vals = plsc.load_gather(
                            rows_v, [ev, jnp.broadcast_to(cv, (16,))])
                        plsc.addupdate_scatter(
                            acc, [rv, jnp.broadcast_to(cv, (16,))], vals)
                    return carry

                lax.fori_loop(0, 2 * H // 4, kb, 0)

        def drain_full(nloc):
            cp = pltpu.async_copy(
                hc_hbm.at[comp_src.at[pl.ds(0, BATCH)]], rows_v, sem)
            cp.wait()
            accumulate(None)
            comp_row[pl.ds(0, 16)] = comp_row[pl.ds(BATCH, 16)]
            comp_src[pl.ds(0, 16)] = comp_src[pl.ds(BATCH, 16)]
            return nloc - BATCH

        for sweep in range(NSWEEP):
            base_row = (sweep * 32 + wid) * OWN

            def z_body(i, carry):
                acc[i // 16, pl.ds((i % 16) * 16, 16)] = zv
                return carry

            lax.fori_loop(0, ACCR * 16, z_body, 0)
            for j in range(10):
                comp_row[pl.ds(j * 16, 16)] = jnp.broadcast_to(DUMMY, (16,))
                comp_src[pl.ds(j * 16, 16)] = jnp.broadcast_to(0, (16,))

            def blk_body(bi, nloc):
                boff = bi * EB
                pltpu.sync_copy(src_hbm.at[pl.ds(boff, EB)], src_blk)
                pltpu.sync_copy(dst_hbm.at[pl.ds(boff, EB)], dst_blk)

                def v_body(vi, nl):
                    off = vi * 16
                    sv = src_blk[pl.ds(off, 16)]
                    dv = dst_blk[pl.ds(off, 16)]
                    tv = plsc.load_gather(ntype_v, [sv])
                    l = tv * T + dv - base_row
                    m = (l >= 0) & (l < OWN)
                    plsc.store_compressed(comp_row.at[pl.ds(nl, 16)], l,
                                          mask=m)
                    plsc.store_compressed(comp_src.at[pl.ds(nl, 16)], sv,
                                          mask=m)
                    nl = nl + jnp.sum(m.astype(jnp.int32))
                    return lax.cond(nl >= BATCH, drain_full, lambda x: x, nl)

                return lax.fori_loop(0, EB // 16, v_body, nloc)

            nloc = lax.fori_loop(0, NBLK, blk_body, 0)
            cp = pltpu.async_copy(
                hc_hbm.at[comp_src.at[pl.ds(0, BATCH)]], rows_v, sem)
            cp.wait()
            accumulate(lambda g: (g * 16 + lanes) < nloc)
            pltpu.sync_copy(acc.at[pl.ds(0, OWN)],
                            out_hbm.at[pl.ds(base_row, OWN)])

    return k(hc, src_p, dst_p, ntype)


def _ln(x, sc, b):
    m = jnp.mean(x, axis=-1, keepdims=True)
    xc = x - m
    v = jnp.mean(xc * xc, axis=-1, keepdims=True)
    return xc * lax.rsqrt(v + 1e-5) * sc + b


def _dot(a, b):
    return jax.lax.dot_general(a, b, (((1,), (0,)), ((), ())),
                               preferred_element_type=jnp.float32)


def _tc_gates(emb_p, acc, W_iou, U_iou, b_iou, W_f, U_f, b_f,
              g0_s, g0_b, g1_s, g1_b, g2_s, g2_b, gf_s, gf_b, gc_s, gc_b):
    acc2 = acc.reshape(2, T, 2 * H)

    def body(emb_ref, a0_ref, a1_ref,
             wiou_ref, uiou_ref, biou_ref, wf_ref, uf_ref, bf_ref,
             g0s, g0b, g1s, g1b, g2s, g2b, gfs, gfb, gcs, gcb,
             h_ref, c_ref):
        emb = emb_ref[...]
        a0 = a0_ref[0]
        a1 = a1_ref[0]
        ht0 = a0[:, :H]
        ct0 = a0[:, H:]
        ht1 = a1[:, :H]
        ct1 = a1[:, H:]
        h_iou = jnp.concatenate([ht0, ht1], axis=1)
        x_f = _dot(emb, wf_ref[...])
        f2 = _dot(h_iou, uf_ref[...])
        bf = bf_ref[...]
        f0 = jax.nn.sigmoid(_ln(x_f + f2[:, :H] + bf, gfs[...], gfb[...]))
        f1 = jax.nn.sigmoid(_ln(x_f + f2[:, H:] + bf, gfs[...], gfb[...]))
        c_cell = f0 * ct0 + f1 * ct1
        iou = (_dot(emb, wiou_ref[...]) + _dot(h_iou, uiou_ref[...])
               + biou_ref[...])
        i_ = jax.nn.sigmoid(_ln(iou[:, :H], g0s[...], g0b[...]))
        o_ = jax.nn.sigmoid(_ln(iou[:, H:2 * H], g1s[...], g1b[...]))
        u_ = jnp.tanh(_ln(iou[:, 2 * H:], g2s[...], g2b[...]))
        c_new = i_ * u_ + c_cell
        h_new = o_ * jnp.tanh(_ln(c_new, gcs[...], gcb[...]))
        h_ref[...] = h_new
        c_ref[...] = c_new

    full = lambda shape: pl.BlockSpec(shape, lambda qq: (0,) * len(shape))
    out = pl.pallas_call(
        body,
        grid=(GRID,),
        in_specs=[
            pl.BlockSpec((R, H), lambda qq: (qq, 0)),
            pl.BlockSpec((1, R, 2 * H), lambda qq: (0, qq, 0)),
            pl.BlockSpec((1, R, 2 * H), lambda qq: (1, qq, 0)),
            full((H, 3 * H)), full((2 * H, 3 * H)), full((1, 3 * H)),
            full((H, H)), full((2 * H, 2 * H)), full((1, H)),
            full((1, H)), full((1, H)), full((1, H)), full((1, H)),
            full((1, H)), full((1, H)), full((1, H)), full((1, H)),
            full((1, H)), full((1, H)),
        ],
        out_specs=[pl.BlockSpec((R, H), lambda qq: (qq, 0)),
                   pl.BlockSpec((R, H), lambda qq: (qq, 0))],
        out_shape=[jax.ShapeDtypeStruct((T, H), jnp.float32),
                   jax.ShapeDtypeStruct((T, H), jnp.float32)],
    )(emb_p, acc2, acc2, W_iou, U_iou, b_iou, W_f, U_f, b_f,
      g0_s, g0_b, g1_s, g1_b, g2_s, g2_b, gf_s, gf_b, gc_s, gc_b)
    return out


def kernel(emb, h, c, W_iou, U_iou, b_iou, W_f, U_f, b_f,
           g0_s, g0_b, g1_s, g1_b, g2_s, g2_b, gf_s, gf_b, gc_s, gc_b,
           n_type, edge_index):
    nt = n_type.astype(jnp.int32)
    ei = edge_index.astype(jnp.int32)
    src = ei[0]
    dst = ei[1]
    e = src.shape[0]
    npad = E_PAD - e
    src_p = jnp.concatenate([src, jnp.zeros((npad,), jnp.int32)])
    pad_dst = N + (jnp.arange(npad, dtype=jnp.int32) % 64)
    dst_p = jnp.concatenate([dst, pad_dst])
    hc = jnp.concatenate([h, c], axis=1)
    acc = _sc_segsum(hc, src_p, dst_p, nt)
    emb_p = jnp.pad(emb, ((0, T - N), (0, 0)))
    r1 = lambda x: x.reshape(1, H)
    h_new, c_new = _tc_gates(
        emb_p, acc, W_iou, U_iou, b_iou, W_f, U_f, b_f,
        r1(g0_s), r1(g0_b), r1(g1_s), r1(g1_b), r1(g2_s), r1(g2_b),
        r1(gf_s), r1(gf_b), r1(gc_s), r1(gc_b))
    return (h_new[:N], c_new[:N])


# speedup vs baseline: 1.6863x; 1.3111x over previous
import functools

import jax
import jax.numpy as jnp
from jax import lax
from jax.experimental import pallas as pl
from jax.experimental.pallas import tpu as pltpu
from jax.experimental.pallas import tpu_sc as plsc

N = 10000
H = 128
T = 10240
ROWS = 2 * T
NSWEEP = 2
OWN = 320
ACCR = OWN + 8
DUMMY = OWN
EB = 1024
NBLK = 314
E_PAD = NBLK * EB
BATCH = 112
CCAP = 1280
R = 2560
GRID = 4


def _sc_segsum(hc, src_p, dst_p, ntype):
    mesh = plsc.VectorSubcoreMesh(core_axis_name="c", subcore_axis_name="s")

    @functools.partial(
        pl.kernel,
        out_type=jax.ShapeDtypeStruct((ROWS, 2 * H), jnp.float32),
        mesh=mesh,
        compiler_params=pltpu.CompilerParams(needs_layout_passes=False),
        scratch_types=[
            pltpu.VMEM((N,), jnp.int32),
            pltpu.VMEM((EB,), jnp.int32),
            pltpu.VMEM((EB,), jnp.int32),
            pltpu.VMEM((EB,), jnp.int32),
            pltpu.VMEM((EB,), jnp.int32),
            pltpu.VMEM((CCAP,), jnp.int32),
            pltpu.VMEM((CCAP,), jnp.int32),
            pltpu.VMEM((BATCH, 2 * H), jnp.float32),
            pltpu.VMEM((ACCR, 2 * H), jnp.float32),
            pltpu.SemaphoreType.DMA,
            pltpu.SemaphoreType.DMA,
            pltpu.SemaphoreType.DMA,
            pltpu.SemaphoreType.DMA,
            pltpu.SemaphoreType.DMA,
        ],
    )
    def k(hc_hbm, src_hbm, dst_hbm, nt_hbm, out_hbm,
          ntype_v, src_blk0, dst_blk0, src_blk1, dst_blk1,
          comp_row, comp_src, rows_v, acc,
          sem, sem_s0, sem_d0, sem_s1, sem_d1):
        c = lax.axis_index("c")
        s = lax.axis_index("s")
        wid = c * 16 + s
        pltpu.sync_copy(nt_hbm, ntype_v)
        lanes = lax.iota(jnp.int32, 16)
        zv = jnp.zeros((16,), jnp.float32)
        bufs = ((src_blk0, dst_blk0, sem_s0, sem_d0),
                (src_blk1, dst_blk1, sem_s1, sem_d1))

        def accumulate(gmask_rows):
            for g in range(BATCH // 16):
                ev = g * 16 + lanes
                rv = comp_row[pl.ds(g * 16, 16)]
                if gmask_rows is not None:
                    rv = jnp.where(gmask_rows(g), rv, DUMMY + (lanes & 7))

                def kb(kk, carry):
                    for u in range(4):
                        cv = kk * 4 + u
                        vals = plsc.load_gather(
                            rows_v, [ev, jnp.broadcast_to(cv, (16,))])
                        plsc.addupdate_scatter(
                            acc, [rv, jnp.broadcast_to(cv, (16,))], vals)
                    return carry

                lax.fori_loop(0, 2 * H // 4, kb, 0)

        def drain_full(nloc):
            cp = pltpu.async_copy(
                hc_hbm.at[comp_src.at[pl.ds(0, BATCH)]], rows_v, sem)
            cp.wait()
            accumulate(None)
            nmove = (nloc - BATCH + 15) // 16

            def mv(t, carry):
                o = t * 16
                comp_row[pl.ds(o, 16)] = comp_row[pl.ds(BATCH + o, 16)]
                comp_src[pl.ds(o, 16)] = comp_src[pl.ds(BATCH + o, 16)]
                return carry

            lax.fori_loop(0, nmove, mv, 0)
            return nloc - BATCH

        for sweep in range(NSWEEP):
            base_row = (sweep * 32 + wid) * OWN

            def z_body(i, carry):
                acc[i // 16, pl.ds((i % 16) * 16, 16)] = zv
                return carry

            lax.fori_loop(0, ACCR * 16, z_body, 0)

            def pf_body(j, carry):
                comp_row[pl.ds(j * 16, 16)] = jnp.broadcast_to(DUMMY, (16,))
                comp_src[pl.ds(j * 16, 16)] = jnp.broadcast_to(0, (16,))
                return carry

            lax.fori_loop(0, CCAP // 16, pf_body, 0)

            for par in range(2):
                sb, db, ss, sd = bufs[par]
                pltpu.async_copy(src_hbm.at[pl.ds(par * EB, EB)], sb, ss)
                pltpu.async_copy(dst_hbm.at[pl.ds(par * EB, EB)], db, sd)

            def outer(bo, nloc):
                for par in range(2):
                    sb, db, ss, sd = bufs[par]
                    bi = bo * 2 + par
                    pltpu.make_async_copy(
                        src_hbm.at[pl.ds(0, EB)], sb, ss).wait()
                    pltpu.make_async_copy(
                        dst_hbm.at[pl.ds(0, EB)], db, sd).wait()

                    def v_body(vi, nl):
                        off = vi * 16
                        sv = sb[pl.ds(off, 16)]
                        dv = db[pl.ds(off, 16)]
                        tv = plsc.load_gather(ntype_v, [sv])
                        l = tv * T + dv - base_row
                        m = (l >= 0) & (l < OWN)
                        plsc.store_compressed(comp_row.at[pl.ds(nl, 16)], l,
                                              mask=m)
                        plsc.store_compressed(comp_src.at[pl.ds(nl, 16)], sv,
                                              mask=m)
                        return nl + jnp.sum(m.astype(jnp.int32))

                    nloc = lax.fori_loop(0, EB // 16, v_body, nloc)

                    @pl.when(bi + 2 < NBLK)
                    def _():
                        off2 = (bi + 2) * EB
                        pltpu.async_copy(src_hbm.at[pl.ds(off2, EB)], sb, ss)
                        pltpu.async_copy(dst_hbm.at[pl.ds(off2, EB)], db, sd)

                    nloc = lax.while_loop(lambda nl: nl >= BATCH,
                                          drain_full, nloc)
                return nloc

            nloc = lax.fori_loop(0, NBLK // 2, outer, 0)
            cp = pltpu.async_copy(
                hc_hbm.at[comp_src.at[pl.ds(0, BATCH)]], rows_v, sem)
            cp.wait()
            accumulate(lambda g: (g * 16 + lanes) < nloc)
            pltpu.sync_copy(acc.at[pl.ds(0, OWN)],
                            out_hbm.at[pl.ds(base_row, OWN)])

    return k(hc, src_p, dst_p, ntype)


def _ln(x, sc, b):
    m = jnp.mean(x, axis=-1, keepdims=True)
    xc = x - m
    v = jnp.mean(xc * xc, axis=-1, keepdims=True)
    return xc * lax.rsqrt(v + 1e-5) * sc + b


def _dot(a, b):
    return jax.lax.dot_general(a, b, (((1,), (0,)), ((), ())),
                               preferred_element_type=jnp.float32)


def _tc_gates(emb_p, acc, W_iou, U_iou, b_iou, W_f, U_f, b_f,
              g0_s, g0_b, g1_s, g1_b, g2_s, g2_b, gf_s, gf_b, gc_s, gc_b):
    acc2 = acc.reshape(2, T, 2 * H)

    def body(emb_ref, a0_ref, a1_ref,
             wiou_ref, uiou_ref, biou_ref, wf_ref, uf_ref, bf_ref,
             g0s, g0b, g1s, g1b, g2s, g2b, gfs, gfb, gcs, gcb,
             h_ref, c_ref):
        emb = emb_ref[...]
        a0 = a0_ref[0]
        a1 = a1_ref[0]
        ht0 = a0[:, :H]
        ct0 = a0[:, H:]
        ht1 = a1[:, :H]
        ct1 = a1[:, H:]
        h_iou = jnp.concatenate([ht0, ht1], axis=1)
        x_f = _dot(emb, wf_ref[...])
        f2 = _dot(h_iou, uf_ref[...])
        bf = bf_ref[...]
        f0 = jax.nn.sigmoid(_ln(x_f + f2[:, :H] + bf, gfs[...], gfb[...]))
        f1 = jax.nn.sigmoid(_ln(x_f + f2[:, H:] + bf, gfs[...], gfb[...]))
        c_cell = f0 * ct0 + f1 * ct1
        iou = (_dot(emb, wiou_ref[...]) + _dot(h_iou, uiou_ref[...])
               + biou_ref[...])
        i_ = jax.nn.sigmoid(_ln(iou[:, :H], g0s[...], g0b[...]))
        o_ = jax.nn.sigmoid(_ln(iou[:, H:2 * H], g1s[...], g1b[...]))
        u_ = jnp.tanh(_ln(iou[:, 2 * H:], g2s[...], g2b[...]))
        c_new = i_ * u_ + c_cell
        h_new = o_ * jnp.tanh(_ln(c_new, gcs[...], gcb[...]))
        h_ref[...] = h_new
        c_ref[...] = c_new

    full = lambda shape: pl.BlockSpec(shape, lambda qq: (0,) * len(shape))
    out = pl.pallas_call(
        body,
        grid=(GRID,),
        in_specs=[
            pl.BlockSpec((R, H), lambda qq: (qq, 0)),
            pl.BlockSpec((1, R, 2 * H), lambda qq: (0, qq, 0)),
            pl.BlockSpec((1, R, 2 * H), lambda qq: (1, qq, 0)),
            full((H, 3 * H)), full((2 * H, 3 * H)), full((1, 3 * H)),
            full((H, H)), full((2 * H, 2 * H)), full((1, H)),
            full((1, H)), full((1, H)), full((1, H)), full((1, H)),
            full((1, H)), full((1, H)), full((1, H)), full((1, H)),
            full((1, H)), full((1, H)),
        ],
        out_specs=[pl.BlockSpec((R, H), lambda qq: (qq, 0)),
                   pl.BlockSpec((R, H), lambda qq: (qq, 0))],
        out_shape=[jax.ShapeDtypeStruct((T, H), jnp.float32),
                   jax.ShapeDtypeStruct((T, H), jnp.float32)],
    )(emb_p, acc2, acc2, W_iou, U_iou, b_iou, W_f, U_f, b_f,
      g0_s, g0_b, g1_s, g1_b, g2_s, g2_b, gf_s, gf_b, gc_s, gc_b)
    return out


def kernel(emb, h, c, W_iou, U_iou, b_iou, W_f, U_f, b_f,
           g0_s, g0_b, g1_s, g1_b, g2_s, g2_b, gf_s, gf_b, gc_s, gc_b,
           n_type, edge_index):
    nt = n_type.astype(jnp.int32)
    ei = edge_index.astype(jnp.int32)
    src = ei[0]
    dst = ei[1]
    e = src.shape[0]
    npad = E_PAD - e
    src_p = jnp.concatenate([src, jnp.zeros((npad,), jnp.int32)])
    pad_dst = N + (jnp.arange(npad, dtype=jnp.int32) % 64)
    dst_p = jnp.concatenate([dst, pad_dst])
    hc = jnp.concatenate([h, c], axis=1)
    acc = _sc_segsum(hc, src_p, dst_p, nt)
    emb_p = jnp.pad(emb, ((0, T - N), (0, 0)))
    r1 = lambda x: x.reshape(1, H)
    h_new, c_new = _tc_gates(
        emb_p, acc, W_iou, U_iou, b_iou, W_f, U_f, b_f,
        r1(g0_s), r1(g0_b), r1(g1_s), r1(g1_b), r1(g2_s), r1(g2_b),
        r1(gf_s), r1(gf_b), r1(gc_s), r1(gc_b))
    return (h_new[:N], c_new[:N])


# speedup vs baseline: 1.7793x; 1.0552x over previous
import functools

import jax
import jax.numpy as jnp
from jax import lax
from jax.experimental import pallas as pl
from jax.experimental.pallas import tpu as pltpu
from jax.experimental.pallas import tpu_sc as plsc

N = 10000
H = 128
T = 10240
ROWS = 2 * T
NSWEEP = 2
OWN = 320
ACCR = OWN + 8
DUMMY = OWN
EB = 2048
NBLK = 158
E_PAD = NBLK * EB
EPT = E_PAD // 32
BATCH = 112
CCAP = 2432
R = 2560
GRID = 4


def _sc_keys(src_p, dst_p, ntype):
    mesh = plsc.VectorSubcoreMesh(core_axis_name="c", subcore_axis_name="s")

    @functools.partial(
        pl.kernel,
        out_type=jax.ShapeDtypeStruct((E_PAD,), jnp.int32),
        mesh=mesh,
        compiler_params=pltpu.CompilerParams(needs_layout_passes=False),
        scratch_types=[
            pltpu.VMEM((N,), jnp.int32),
            pltpu.VMEM((EPT,), jnp.int32),
            pltpu.VMEM((EPT,), jnp.int32),
        ],
    )
    def ka(src_hbm, dst_hbm, nt_hbm, key_hbm, ntype_v, src_v, key_v):
        c = lax.axis_index("c")
        s = lax.axis_index("s")
        base_e = (c * 16 + s) * EPT
        pltpu.sync_copy(nt_hbm, ntype_v)
        pltpu.sync_copy(src_hbm.at[pl.ds(base_e, EPT)], src_v)
        pltpu.sync_copy(dst_hbm.at[pl.ds(base_e, EPT)], key_v)

        def body(i, carry):
            for u in range(4):
                off = (i * 4 + u) * 16
                sv = src_v[pl.ds(off, 16)]
                tv = plsc.load_gather(ntype_v, [sv])
                key_v[pl.ds(off, 16)] = tv * T + key_v[pl.ds(off, 16)]
            return carry

        lax.fori_loop(0, EPT // 64, body, 0)
        pltpu.sync_copy(key_v, key_hbm.at[pl.ds(base_e, EPT)])

    return ka(src_p, dst_p, ntype)


def _sc_segsum(hc, src_p, key_p):
    mesh = plsc.VectorSubcoreMesh(core_axis_name="c", subcore_axis_name="s")

    @functools.partial(
        pl.kernel,
        out_type=jax.ShapeDtypeStruct((ROWS, 2 * H), jnp.float32),
        mesh=mesh,
        compiler_params=pltpu.CompilerParams(needs_layout_passes=False),
        scratch_types=[
            pltpu.VMEM((EB,), jnp.int32),
            pltpu.VMEM((EB,), jnp.int32),
            pltpu.VMEM((EB,), jnp.int32),
            pltpu.VMEM((EB,), jnp.int32),
            pltpu.VMEM((CCAP,), jnp.int32),
            pltpu.VMEM((CCAP,), jnp.int32),
            pltpu.VMEM((BATCH, 2 * H), jnp.float32),
            pltpu.VMEM((ACCR, 2 * H), jnp.float32),
            pltpu.SemaphoreType.DMA,
            pltpu.SemaphoreType.DMA,
            pltpu.SemaphoreType.DMA,
            pltpu.SemaphoreType.DMA,
            pltpu.SemaphoreType.DMA,
        ],
    )
    def k(hc_hbm, src_hbm, key_hbm, out_hbm,
          src_blk0, key_blk0, src_blk1, key_blk1,
          comp_row, comp_src, rows_v, acc,
          sem, sem_s0, sem_d0, sem_s1, sem_d1):
        c = lax.axis_index("c")
        s = lax.axis_index("s")
        wid = c * 16 + s
        lanes = lax.iota(jnp.int32, 16)
        zv = jnp.zeros((16,), jnp.float32)
        bufs = ((src_blk0, key_blk0, sem_s0, sem_d0),
                (src_blk1, key_blk1, sem_s1, sem_d1))

        def accumulate(gmask_rows):
            for g in range(BATCH // 16):
                ev = g * 16 + lanes
                rv = comp_row[pl.ds(g * 16, 16)]
                if gmask_rows is not None:
                    rv = jnp.where(gmask_rows(g), rv, DUMMY + (lanes & 7))

                def kb(kk, carry):
                    for u in range(8):
                        cv = kk * 8 + u
                        vals = plsc.load_gather(
                            rows_v, [ev, jnp.broadcast_to(cv, (16,))])
                        plsc.addupdate_scatter(
                            acc, [rv, jnp.broadcast_to(cv, (16,))], vals)
                    return carry

                lax.fori_loop(0, 2 * H // 8, kb, 0)

        def drain_full(nloc):
            cp = pltpu.async_copy(
                hc_hbm.at[comp_src.at[pl.ds(0, BATCH)]], rows_v, sem)
            cp.wait()
            accumulate(None)
            nmove = (nloc - BATCH + 15) // 16

            def mv(t, carry):
                o = t * 16
                comp_row[pl.ds(o, 16)] = comp_row[pl.ds(BATCH + o, 16)]
                comp_src[pl.ds(o, 16)] = comp_src[pl.ds(BATCH + o, 16)]
                return carry

            lax.fori_loop(0, nmove, mv, 0)
            return nloc - BATCH

        for sweep in range(NSWEEP):
            base_row = (sweep * 32 + wid) * OWN

            def z_body(r, carry):
                for u in range(16):
                    acc[r, pl.ds(u * 16, 16)] = zv
                return carry

            lax.fori_loop(0, ACCR, z_body, 0)

            def pf_body(j, carry):
                comp_row[pl.ds(j * 16, 16)] = jnp.broadcast_to(DUMMY, (16,))
                comp_src[pl.ds(j * 16, 16)] = jnp.broadcast_to(0, (16,))
                return carry

            lax.fori_loop(0, CCAP // 16, pf_body, 0)

            for par in range(2):
                sb, kb_, ss, sd = bufs[par]
                pltpu.async_copy(src_hbm.at[pl.ds(par * EB, EB)], sb, ss)
                pltpu.async_copy(key_hbm.at[pl.ds(par * EB, EB)], kb_, sd)

            def outer(bo, nloc):
                for par in range(2):
                    sb, kb_, ss, sd = bufs[par]
                    bi = bo * 2 + par
                    pltpu.make_async_copy(
                        src_hbm.at[pl.ds(0, EB)], sb, ss).wait()
                    pltpu.make_async_copy(
                        key_hbm.at[pl.ds(0, EB)], kb_, sd).wait()

                    def v_body(vi, nl):
                        for u in range(4):
                            off = vi * 64 + u * 16
                            kv = kb_[pl.ds(off, 16)]
                            sv = sb[pl.ds(off, 16)]
                            l = kv - base_row
                            m = (l >= 0) & (l < OWN)
                            plsc.store_compressed(
                                comp_row.at[pl.ds(nl, 16)], l, mask=m)
                            plsc.store_compressed(
                                comp_src.at[pl.ds(nl, 16)], sv, mask=m)
                            pc = plsc.all_reduce_population_count(m)
                            nl = nl + pc[0]
                        return nl

                    nloc = lax.fori_loop(0, EB // 64, v_body, nloc)

                    @pl.when(bi + 2 < NBLK)
                    def _():
                        off2 = (bi + 2) * EB
                        pltpu.async_copy(src_hbm.at[pl.ds(off2, EB)], sb, ss)
                        pltpu.async_copy(key_hbm.at[pl.ds(off2, EB)], kb_, sd)

                    nloc = lax.while_loop(lambda nl: nl >= BATCH,
                                          drain_full, nloc)
                return nloc

            nloc = lax.fori_loop(0, NBLK // 2, outer, 0)
            cp = pltpu.async_copy(
                hc_hbm.at[comp_src.at[pl.ds(0, BATCH)]], rows_v, sem)
            cp.wait()
            accumulate(lambda g: (g * 16 + lanes) < nloc)
            pltpu.sync_copy(acc.at[pl.ds(0, OWN)],
                            out_hbm.at[pl.ds(base_row, OWN)])

    return k(hc, src_p, key_p)


def _ln(x, sc, b):
    m = jnp.mean(x, axis=-1, keepdims=True)
    xc = x - m
    v = jnp.mean(xc * xc, axis=-1, keepdims=True)
    return xc * lax.rsqrt(v + 1e-5) * sc + b


def _dot(a, b):
    return jax.lax.dot_general(a, b, (((1,), (0,)), ((), ())),
                               preferred_element_type=jnp.float32)


def _tc_gates(emb_p, acc, W_iou, U_iou, b_iou, W_f, U_f, b_f,
              g0_s, g0_b, g1_s, g1_b, g2_s, g2_b, gf_s, gf_b, gc_s, gc_b):
    acc2 = acc.reshape(2, T, 2 * H)

    def body(emb_ref, a0_ref, a1_ref,
             wiou_ref, uiou_ref, biou_ref, wf_ref, uf_ref, bf_ref,
             g0s, g0b, g1s, g1b, g2s, g2b, gfs, gfb, gcs, gcb,
             h_ref, c_ref):
        emb = emb_ref[...]
        a0 = a0_ref[0]
        a1 = a1_ref[0]
        ht0 = a0[:, :H]
        ct0 = a0[:, H:]
        ht1 = a1[:, :H]
        ct1 = a1[:, H:]
        h_iou = jnp.concatenate([ht0, ht1], axis=1)
        x_f = _dot(emb, wf_ref[...])
        f2 = _dot(h_iou, uf_ref[...])
        bf = bf_ref[...]
        f0 = jax.nn.sigmoid(_ln(x_f + f2[:, :H] + bf, gfs[...], gfb[...]))
        f1 = jax.nn.sigmoid(_ln(x_f + f2[:, H:] + bf, gfs[...], gfb[...]))
        c_cell = f0 * ct0 + f1 * ct1
        iou = (_dot(emb, wiou_ref[...]) + _dot(h_iou, uiou_ref[...])
               + biou_ref[...])
        i_ = jax.nn.sigmoid(_ln(iou[:, :H], g0s[...], g0b[...]))
        o_ = jax.nn.sigmoid(_ln(iou[:, H:2 * H], g1s[...], g1b[...]))
        u_ = jnp.tanh(_ln(iou[:, 2 * H:], g2s[...], g2b[...]))
        c_new = i_ * u_ + c_cell
        h_new = o_ * jnp.tanh(_ln(c_new, gcs[...], gcb[...]))
        h_ref[...] = h_new
        c_ref[...] = c_new

    full = lambda shape: pl.BlockSpec(shape, lambda qq: (0,) * len(shape))
    out = pl.pallas_call(
        body,
        grid=(GRID,),
        in_specs=[
            pl.BlockSpec((R, H), lambda qq: (qq, 0)),
            pl.BlockSpec((1, R, 2 * H), lambda qq: (0, qq, 0)),
            pl.BlockSpec((1, R, 2 * H), lambda qq: (1, qq, 0)),
            full((H, 3 * H)), full((2 * H, 3 * H)), full((1, 3 * H)),
            full((H, H)), full((2 * H, 2 * H)), full((1, H)),
            full((1, H)), full((1, H)), full((1, H)), full((1, H)),
            full((1, H)), full((1, H)), full((1, H)), full((1, H)),
            full((1, H)), full((1, H)),
        ],
        out_specs=[pl.BlockSpec((R, H), lambda qq: (qq, 0)),
                   pl.BlockSpec((R, H), lambda qq: (qq, 0))],
        out_shape=[jax.ShapeDtypeStruct((T, H), jnp.float32),
                   jax.ShapeDtypeStruct((T, H), jnp.float32)],
    )(emb_p, acc2, acc2, W_iou, U_iou, b_iou, W_f, U_f, b_f,
      g0_s, g0_b, g1_s, g1_b, g2_s, g2_b, gf_s, gf_b, gc_s, gc_b)
    return out


def kernel(emb, h, c, W_iou, U_iou, b_iou, W_f, U_f, b_f,
           g0_s, g0_b, g1_s, g1_b, g2_s, g2_b, gf_s, gf_b, gc_s, gc_b,
           n_type, edge_index):
    nt = n_type.astype(jnp.int32)
    ei = edge_index.astype(jnp.int32)
    src = ei[0]
    dst = ei[1]
    e = src.shape[0]
    npad = E_PAD - e
    src_p = jnp.concatenate([src, jnp.zeros((npad,), jnp.int32)])
    pad_dst = N + (jnp.arange(npad, dtype=jnp.int32) % 64)
    dst_p = jnp.concatenate([dst, pad_dst])
    hc = jnp.concatenate([h, c], axis=1)
    keys = _sc_keys(src_p, dst_p, nt)
    acc = _sc_segsum(hc, src_p, keys)
    emb_p = jnp.pad(emb, ((0, T - N), (0, 0)))
    r1 = lambda x: x.reshape(1, H)
    h_new, c_new = _tc_gates(
        emb_p, acc, W_iou, U_iou, b_iou, W_f, U_f, b_f,
        r1(g0_s), r1(g0_b), r1(g1_s), r1(g1_b), r1(g2_s), r1(g2_b),
        r1(gf_s), r1(gf_b), r1(gc_s), r1(gc_b))
    return (h_new[:N], c_new[:N])


# speedup vs baseline: 5.0376x; 2.8312x over previous
import functools

import jax
import jax.numpy as jnp
from jax import lax
from jax.experimental import pallas as pl
from jax.experimental.pallas import tpu as pltpu
from jax.experimental.pallas import tpu_sc as plsc

N = 10000
H = 128
T = 10240
ROWS = 2 * T
NSWEEP = 2
OWN = 320
ACCR = OWN + 8
DUMMY = OWN
EB = 2048
NBLK = 158
E_PAD = NBLK * EB
EPT = E_PAD // 32
BATCH = 128
CCAP = 2432
R = 2560
GRID = 4


def _sc_keys(src_p, dst_p, ntype):
    mesh = plsc.VectorSubcoreMesh(core_axis_name="c", subcore_axis_name="s")

    @functools.partial(
        pl.kernel,
        out_type=jax.ShapeDtypeStruct((E_PAD,), jnp.int32),
        mesh=mesh,
        compiler_params=pltpu.CompilerParams(needs_layout_passes=False),
        scratch_types=[
            pltpu.VMEM((N,), jnp.int32),
            pltpu.VMEM((EPT,), jnp.int32),
            pltpu.VMEM((EPT,), jnp.int32),
        ],
    )
    def ka(src_hbm, dst_hbm, nt_hbm, key_hbm, ntype_v, src_v, key_v):
        c = lax.axis_index("c")
        s = lax.axis_index("s")
        base_e = (c * 16 + s) * EPT
        pltpu.sync_copy(nt_hbm, ntype_v)
        pltpu.sync_copy(src_hbm.at[pl.ds(base_e, EPT)], src_v)
        pltpu.sync_copy(dst_hbm.at[pl.ds(base_e, EPT)], key_v)

        def body(i, carry):
            for u in range(4):
                off = (i * 4 + u) * 16
                sv = src_v[pl.ds(off, 16)]
                tv = plsc.load_gather(ntype_v, [sv])
                key_v[pl.ds(off, 16)] = tv * T + key_v[pl.ds(off, 16)]
            return carry

        lax.fori_loop(0, EPT // 64, body, 0)
        pltpu.sync_copy(key_v, key_hbm.at[pl.ds(base_e, EPT)])

    return ka(src_p, dst_p, ntype)


def _sc_segsum(hc, src_p, key_p):
    mesh = plsc.VectorSubcoreMesh(core_axis_name="c", subcore_axis_name="s")

    @functools.partial(
        pl.kernel,
        out_type=jax.ShapeDtypeStruct((ROWS, 2 * H), jnp.float32),
        mesh=mesh,
        compiler_params=pltpu.CompilerParams(needs_layout_passes=False),
        scratch_types=[
            pltpu.VMEM((EB,), jnp.int32),
            pltpu.VMEM((EB,), jnp.int32),
            pltpu.VMEM((EB,), jnp.int32),
            pltpu.VMEM((EB,), jnp.int32),
            pltpu.VMEM((CCAP,), jnp.int32),
            pltpu.VMEM((CCAP,), jnp.int32),
            pltpu.VMEM((BATCH, 2 * H), jnp.float32),
            pltpu.VMEM((ACCR, 2 * H), jnp.float32),
            pltpu.SemaphoreType.DMA,
            pltpu.SemaphoreType.DMA,
            pltpu.SemaphoreType.DMA,
            pltpu.SemaphoreType.DMA,
            pltpu.SemaphoreType.DMA,
        ],
    )
    def k(hc_hbm, src_hbm, key_hbm, out_hbm,
          src_blk0, key_blk0, src_blk1, key_blk1,
          comp_row, comp_src, rows_v, acc,
          sem, sem_s0, sem_d0, sem_s1, sem_d1):
        c = lax.axis_index("c")
        s = lax.axis_index("s")
        wid = c * 16 + s
        lanes = lax.iota(jnp.int32, 16)
        zv = jnp.zeros((16,), jnp.float32)
        bufs = ((src_blk0, key_blk0, sem_s0, sem_d0),
                (src_blk1, key_blk1, sem_s1, sem_d1))

        def accumulate(limit):
            def g_body(g, carry):
                rv = comp_row[pl.ds(g * 16, 16)]
                for lane in range(16):
                    r = rv[lane]
                    if limit is not None:
                        r = jnp.where(g * 16 + lane < limit, r,
                                      DUMMY + (lane & 7))
                    e = g * 16 + lane
                    for u in range(16):
                        sl = pl.ds(u * 16, 16)
                        acc[r, sl] = acc[r, sl] + rows_v[e, sl]
                return carry

            lax.fori_loop(0, BATCH // 16, g_body, 0)

        def drain_full(nloc):
            cp = pltpu.async_copy(
                hc_hbm.at[comp_src.at[pl.ds(0, BATCH)]], rows_v, sem)
            cp.wait()
            accumulate(None)
            nmove = (nloc - BATCH + 15) // 16

            def mv(t, carry):
                o = t * 16
                comp_row[pl.ds(o, 16)] = comp_row[pl.ds(BATCH + o, 16)]
                comp_src[pl.ds(o, 16)] = comp_src[pl.ds(BATCH + o, 16)]
                return carry

            lax.fori_loop(0, nmove, mv, 0)
            return nloc - BATCH

        for sweep in range(NSWEEP):
            base_row = (sweep * 32 + wid) * OWN

            def z_body(r, carry):
                for u in range(16):
                    acc[r, pl.ds(u * 16, 16)] = zv
                return carry

            lax.fori_loop(0, ACCR, z_body, 0)

            def pf_body(j, carry):
                comp_row[pl.ds(j * 16, 16)] = jnp.broadcast_to(DUMMY, (16,))
                comp_src[pl.ds(j * 16, 16)] = jnp.broadcast_to(0, (16,))
                return carry

            lax.fori_loop(0, CCAP // 16, pf_body, 0)

            for par in range(2):
                sb, kb_, ss, sd = bufs[par]
                pltpu.async_copy(src_hbm.at[pl.ds(par * EB, EB)], sb, ss)
                pltpu.async_copy(key_hbm.at[pl.ds(par * EB, EB)], kb_, sd)

            def outer(bo, nloc):
                for par in range(2):
                    sb, kb_, ss, sd = bufs[par]
                    bi = bo * 2 + par
                    pltpu.make_async_copy(
                        src_hbm.at[pl.ds(0, EB)], sb, ss).wait()
                    pltpu.make_async_copy(
                        key_hbm.at[pl.ds(0, EB)], kb_, sd).wait()

                    def v_body(vi, nl):
                        for u in range(4):
                            off = vi * 64 + u * 16
                            kv = kb_[pl.ds(off, 16)]
                            sv = sb[pl.ds(off, 16)]
                            l = kv - base_row
                            m = (l >= 0) & (l < OWN)
                            plsc.store_compressed(
                                comp_row.at[pl.ds(nl, 16)], l, mask=m)
                            plsc.store_compressed(
                                comp_src.at[pl.ds(nl, 16)], sv, mask=m)
                            pc = plsc.all_reduce_population_count(m)
                            nl = nl + pc[0]
                        return nl

                    nloc = lax.fori_loop(0, EB // 64, v_body, nloc)

                    @pl.when(bi + 2 < NBLK)
                    def _():
                        off2 = (bi + 2) * EB
                        pltpu.async_copy(src_hbm.at[pl.ds(off2, EB)], sb, ss)
                        pltpu.async_copy(key_hbm.at[pl.ds(off2, EB)], kb_, sd)

                    nloc = lax.while_loop(lambda nl: nl >= BATCH,
                                          drain_full, nloc)
                return nloc

            nloc = lax.fori_loop(0, NBLK // 2, outer, 0)
            cp = pltpu.async_copy(
                hc_hbm.at[comp_src.at[pl.ds(0, BATCH)]], rows_v, sem)
            cp.wait()
            accumulate(nloc)
            pltpu.sync_copy(acc.at[pl.ds(0, OWN)],
                            out_hbm.at[pl.ds(base_row, OWN)])

    return k(hc, src_p, key_p)


def _ln(x, sc, b):
    m = jnp.mean(x, axis=-1, keepdims=True)
    xc = x - m
    v = jnp.mean(xc * xc, axis=-1, keepdims=True)
    return xc * lax.rsqrt(v + 1e-5) * sc + b


def _dot(a, b):
    return jax.lax.dot_general(a, b, (((1,), (0,)), ((), ())),
                               preferred_element_type=jnp.float32)


def _tc_gates(emb_p, acc, W_iou, U_iou, b_iou, W_f, U_f, b_f,
              g0_s, g0_b, g1_s, g1_b, g2_s, g2_b, gf_s, gf_b, gc_s, gc_b):
    acc2 = acc.reshape(2, T, 2 * H)

    def body(emb_ref, a0_ref, a1_ref,
             wiou_ref, uiou_ref, biou_ref, wf_ref, uf_ref, bf_ref,
             g0s, g0b, g1s, g1b, g2s, g2b, gfs, gfb, gcs, gcb,
             h_ref, c_ref):
        emb = emb_ref[...]
        a0 = a0_ref[0]
        a1 = a1_ref[0]
        ht0 = a0[:, :H]
        ct0 = a0[:, H:]
        ht1 = a1[:, :H]
        ct1 = a1[:, H:]
        h_iou = jnp.concatenate([ht0, ht1], axis=1)
        x_f = _dot(emb, wf_ref[...])
        f2 = _dot(h_iou, uf_ref[...])
        bf = bf_ref[...]
        f0 = jax.nn.sigmoid(_ln(x_f + f2[:, :H] + bf, gfs[...], gfb[...]))
        f1 = jax.nn.sigmoid(_ln(x_f + f2[:, H:] + bf, gfs[...], gfb[...]))
        c_cell = f0 * ct0 + f1 * ct1
        iou = (_dot(emb, wiou_ref[...]) + _dot(h_iou, uiou_ref[...])
               + biou_ref[...])
        i_ = jax.nn.sigmoid(_ln(iou[:, :H], g0s[...], g0b[...]))
        o_ = jax.nn.sigmoid(_ln(iou[:, H:2 * H], g1s[...], g1b[...]))
        u_ = jnp.tanh(_ln(iou[:, 2 * H:], g2s[...], g2b[...]))
        c_new = i_ * u_ + c_cell
        h_new = o_ * jnp.tanh(_ln(c_new, gcs[...], gcb[...]))
        h_ref[...] = h_new
        c_ref[...] = c_new

    full = lambda shape: pl.BlockSpec(shape, lambda qq: (0,) * len(shape))
    out = pl.pallas_call(
        body,
        grid=(GRID,),
        in_specs=[
            pl.BlockSpec((R, H), lambda qq: (qq, 0)),
            pl.BlockSpec((1, R, 2 * H), lambda qq: (0, qq, 0)),
            pl.BlockSpec((1, R, 2 * H), lambda qq: (1, qq, 0)),
            full((H, 3 * H)), full((2 * H, 3 * H)), full((1, 3 * H)),
            full((H, H)), full((2 * H, 2 * H)), full((1, H)),
            full((1, H)), full((1, H)), full((1, H)), full((1, H)),
            full((1, H)), full((1, H)), full((1, H)), full((1, H)),
            full((1, H)), full((1, H)),
        ],
        out_specs=[pl.BlockSpec((R, H), lambda qq: (qq, 0)),
                   pl.BlockSpec((R, H), lambda qq: (qq, 0))],
        out_shape=[jax.ShapeDtypeStruct((T, H), jnp.float32),
                   jax.ShapeDtypeStruct((T, H), jnp.float32)],
    )(emb_p, acc2, acc2, W_iou, U_iou, b_iou, W_f, U_f, b_f,
      g0_s, g0_b, g1_s, g1_b, g2_s, g2_b, gf_s, gf_b, gc_s, gc_b)
    return out


def kernel(emb, h, c, W_iou, U_iou, b_iou, W_f, U_f, b_f,
           g0_s, g0_b, g1_s, g1_b, g2_s, g2_b, gf_s, gf_b, gc_s, gc_b,
           n_type, edge_index):
    nt = n_type.astype(jnp.int32)
    ei = edge_index.astype(jnp.int32)
    src = ei[0]
    dst = ei[1]
    e = src.shape[0]
    npad = E_PAD - e
    src_p = jnp.concatenate([src, jnp.zeros((npad,), jnp.int32)])
    pad_dst = N + (jnp.arange(npad, dtype=jnp.int32) % 64)
    dst_p = jnp.concatenate([dst, pad_dst])
    hc = jnp.concatenate([h, c], axis=1)
    keys = _sc_keys(src_p, dst_p, nt)
    acc = _sc_segsum(hc, src_p, keys)
    emb_p = jnp.pad(emb, ((0, T - N), (0, 0)))
    r1 = lambda x: x.reshape(1, H)
    h_new, c_new = _tc_gates(
        emb_p, acc, W_iou, U_iou, b_iou, W_f, U_f, b_f,
        r1(g0_s), r1(g0_b), r1(g1_s), r1(g1_b), r1(g2_s), r1(g2_b),
        r1(gf_s), r1(gf_b), r1(gc_s), r1(gc_b))
    return (h_new[:N], c_new[:N])
